# Initial kernel scaffold; baseline (speedup 1.0000x reference)
import jax
import jax.numpy as jnp
from jax.experimental import pallas as pl


def kernel(x, Wr, bias, Wg, Wu, Wd, Wg_s, Wu_s, Wd_s):
    raise NotImplementedError("write your pallas kernel here")


# speedup vs baseline: 4.8578x; 4.8578x over previous
import functools

import jax
import jax.numpy as jnp
from jax.experimental import pallas as pl
from jax.experimental.pallas import tpu as pltpu

_INTERPRET = False

T = 2048
H = 1024
M = 512
E = 64
BT = 128
NB = 96
NSLOT = NB * BT
NTB = T // BT


def _sigmoid(x):
    return 1.0 / (1.0 + jnp.exp(-x))


def _top2_cols(probs_t):
    sub = jax.lax.broadcasted_iota(jnp.int32, probs_t.shape, 0)
    m0 = jnp.max(probs_t, axis=0, keepdims=True)
    i0 = jnp.min(jnp.where(probs_t == m0, sub, jnp.int32(E)), axis=0, keepdims=True)
    p1 = jnp.where(sub == i0, -1.0, probs_t)
    m1 = jnp.max(p1, axis=0, keepdims=True)
    i1 = jnp.min(jnp.where(p1 == m1, sub, jnp.int32(E)), axis=0, keepdims=True)
    return m0, m1, i0, i1


def _router_kernel(x_ref, wr_ref, b_ref, pos_ref, be_ref):
    x = x_ref[...]
    wr = wr_ref[...]
    logits_t = jax.lax.dot_general(wr, x, (((1,), (1,)), ((), ())),
                                   preferred_element_type=jnp.float32)
    logits_t = logits_t + b_ref[...]
    probs_t = _sigmoid(logits_t)
    _, _, i0, i1 = _top2_cols(probs_t)

    sub = jax.lax.broadcasted_iota(jnp.int32, (E, T), 0)
    oh0 = (sub == i0).astype(jnp.float32)
    oh1 = (sub == i1).astype(jnp.float32)

    lane = jax.lax.broadcasted_iota(jnp.int32, (BT, BT), 1)
    row = jax.lax.broadcasted_iota(jnp.int32, (BT, BT), 0)
    tri_incl = (row <= lane).astype(jnp.float32)
    excl_chunks0, excl_chunks1 = [], []
    carry0 = jnp.zeros((E, 1), jnp.float32)
    carry1 = jnp.zeros((E, 1), jnp.float32)
    for c in range(T // BT):
        ch0 = oh0[:, c * BT:(c + 1) * BT]
        ch1 = oh1[:, c * BT:(c + 1) * BT]
        incl0 = jax.lax.dot_general(ch0, tri_incl, (((1,), (0,)), ((), ())),
                                    preferred_element_type=jnp.float32) + carry0
        incl1 = jax.lax.dot_general(ch1, tri_incl, (((1,), (0,)), ((), ())),
                                    preferred_element_type=jnp.float32) + carry1
        excl_chunks0.append(incl0 - ch0)
        excl_chunks1.append(incl1 - ch1)
        carry0 = incl0[:, BT - 1:BT]
        carry1 = incl1[:, BT - 1:BT]
    excl0 = jnp.concatenate(excl_chunks0, axis=1)
    excl1 = jnp.concatenate(excl_chunks1, axis=1)
    count0 = carry0
    counts = carry0 + carry1

    pc = (((counts.astype(jnp.int32) + (BT - 1)) // BT) * BT).astype(jnp.float32)
    er = jax.lax.broadcasted_iota(jnp.int32, (E, E), 0)
    ec = jax.lax.broadcasted_iota(jnp.int32, (E, E), 1)
    tri_excl = (er > ec).astype(jnp.float32)
    off = jax.lax.dot_general(tri_excl, pc, (((1,), (0,)), ((), ())),
                              preferred_element_type=jnp.float32)

    pos0 = jnp.sum(off * oh0 + excl0 * oh0, axis=0, keepdims=True)
    pos1 = jnp.sum((off + count0) * oh1 + excl1 * oh1, axis=0, keepdims=True)

    sub8 = jax.lax.broadcasted_iota(jnp.int32, (8, T), 0)
    pos_ref[...] = jnp.where(sub8 == 0, pos0, jnp.where(sub8 == 1, pos1, 0.0)
                             ).astype(jnp.int32)

    ends = off + pc
    bcol = jax.lax.broadcasted_iota(jnp.int32, (E, 128), 1).astype(jnp.float32) * BT
    bexp = jnp.sum((ends <= bcol).astype(jnp.float32), axis=0, keepdims=True)
    bexp = jnp.minimum(bexp, float(E - 1))
    sub8b = jax.lax.broadcasted_iota(jnp.int32, (8, 128), 0)
    be_ref[...] = jnp.where(sub8b == 0, bexp, 0.0).astype(jnp.int32)


def _router(x_flat, wr, bias):
    return pl.pallas_call(
        _router_kernel,
        out_shape=(jax.ShapeDtypeStruct((8, T), jnp.int32),
                   jax.ShapeDtypeStruct((8, 128), jnp.int32)),
        interpret=_INTERPRET,
    )(x_flat, wr, bias.reshape(E, 1))


def _expert_block(x, wg, wu, wd):
    g = jax.lax.dot_general(x, wg, (((1,), (1,)), ((), ())),
                            preferred_element_type=jnp.float32)
    u = jax.lax.dot_general(x, wu, (((1,), (1,)), ((), ())),
                            preferred_element_type=jnp.float32)
    a = g * _sigmoid(g) * u
    return jax.lax.dot_general(a, wd, (((1,), (1,)), ((), ())),
                               preferred_element_type=jnp.float32)


def _gmm_kernel(be_ref, xs_ref, wg_ref, wu_ref, wd_ref, ys_ref):
    del be_ref
    ys_ref[...] = _expert_block(xs_ref[...], wg_ref[0], wu_ref[0], wd_ref[0])


def _grouped_mlp(be, x_sorted, wg, wu, wd):
    grid_spec = pltpu.PrefetchScalarGridSpec(
        num_scalar_prefetch=1,
        grid=(NB,),
        in_specs=[
            pl.BlockSpec((BT, H), lambda b, be: (b, 0)),
            pl.BlockSpec((1, M, H), lambda b, be: (be[0, b], 0, 0)),
            pl.BlockSpec((1, M, H), lambda b, be: (be[0, b], 0, 0)),
            pl.BlockSpec((1, H, M), lambda b, be: (be[0, b], 0, 0)),
        ],
        out_specs=pl.BlockSpec((BT, H), lambda b, be: (b, 0)),
    )
    return pl.pallas_call(
        _gmm_kernel,
        grid_spec=grid_spec,
        out_shape=jax.ShapeDtypeStruct((NSLOT, H), jnp.float32),
        interpret=_INTERPRET,
    )(be, x_sorted, wg, wu, wd)


def _combine_kernel(x_ref, y0_ref, y1_ref, wr_ref, b_ref,
                    wgs_ref, wus_ref, wds_ref, o_ref):
    x = x_ref[...]
    logits = jax.lax.dot_general(x, wr_ref[...], (((1,), (1,)), ((), ())),
                                 preferred_element_type=jnp.float32)
    logits = logits + b_ref[...]
    probs = _sigmoid(logits)
    lane = jax.lax.broadcasted_iota(jnp.int32, probs.shape, 1)
    m0 = jnp.max(probs, axis=1, keepdims=True)
    i0 = jnp.min(jnp.where(probs == m0, lane, jnp.int32(E)), axis=1, keepdims=True)
    p1 = jnp.where(lane == i0, -1.0, probs)
    m1 = jnp.max(p1, axis=1, keepdims=True)
    denom = m0 + m1 + 1e-9
    w0 = m0 / denom
    w1 = m1 / denom
    shared = _expert_block(x, wgs_ref[...], wus_ref[...], wds_ref[...])
    o_ref[...] = w0 * y0_ref[...] + w1 * y1_ref[...] + shared


def _combine(x_flat, y0, y1, wr, bias, wgs, wus, wds):
    return pl.pallas_call(
        _combine_kernel,
        grid=(NTB,),
        in_specs=[
            pl.BlockSpec((BT, H), lambda t: (t, 0)),
            pl.BlockSpec((BT, H), lambda t: (t, 0)),
            pl.BlockSpec((BT, H), lambda t: (t, 0)),
            pl.BlockSpec((E, H), lambda t: (0, 0)),
            pl.BlockSpec((1, E), lambda t: (0, 0)),
            pl.BlockSpec((M, H), lambda t: (0, 0)),
            pl.BlockSpec((M, H), lambda t: (0, 0)),
            pl.BlockSpec((H, M), lambda t: (0, 0)),
        ],
        out_specs=pl.BlockSpec((BT, H), lambda t: (t, 0)),
        out_shape=jax.ShapeDtypeStruct((T, H), jnp.float32),
        interpret=_INTERPRET,
    )(x_flat, y0, y1, wr, bias.reshape(1, E), wgs, wus, wds)


def kernel(x, Wr, bias, Wg, Wu, Wd, Wg_s, Wu_s, Wd_s):
    b, s, h = x.shape
    x_flat = x.reshape(b * s, h)

    pos, be = _router(x_flat, Wr, bias)
    pos0 = pos[0]
    pos1 = pos[1]

    tok = jnp.arange(T, dtype=jnp.int32)
    token_for_slot = jnp.zeros((NSLOT,), jnp.int32).at[pos0].set(tok).at[pos1].set(tok)
    x_sorted = x_flat[token_for_slot]

    y_sorted = _grouped_mlp(be, x_sorted, Wg, Wu, Wd)

    y0 = y_sorted[pos0]
    y1 = y_sorted[pos1]

    out = _combine(x_flat, y0, y1, Wr, bias, Wg_s, Wu_s, Wd_s)
    return out.reshape(b, s, h)
